# Initial kernel scaffold; baseline (speedup 1.0000x reference)
import jax
import jax.numpy as jnp
from jax.experimental import pallas as pl


def kernel(x, edge_index, W, a):
    raise NotImplementedError("write your pallas kernel here")


# speedup vs baseline: 1.4228x; 1.4228x over previous
import functools

import jax
import jax.numpy as jnp
from jax import lax
from jax.experimental import pallas as pl
from jax.experimental.pallas import tpu as pltpu
from jax.experimental.pallas import tpu_sc as plsc

N_NODES = 10000
F = 128
FH = 64
N_EDGES = 320000
E_TOT = N_EDGES + N_NODES
CHUNK = 240
CHUNKS = E_TOT // CHUNK
K_ITERS = -(-CHUNKS // 16)
N_PAD = 10240
STRIDE = 640
NH = 5120
ROWS_T = NH // 16
AGG_ROWS = NH + 8


def _mm_body(x_ref, w_ref, a2_ref, act_ref, sp_ref):
    act = lax.dot_general(x_ref[...], w_ref[...], (((1,), (1,)), ((), ())),
                          preferred_element_type=jnp.float32)
    act_ref[...] = act
    sp_ref[...] = lax.dot_general(act, a2_ref[...], (((1,), (0,)), ((), ())),
                                  preferred_element_type=jnp.float32)


def _node_stage(x, W, a2):
    grid = N_NODES // 1000
    return pl.pallas_call(
        _mm_body,
        grid=(grid,),
        in_specs=[
            pl.BlockSpec((1000, F), lambda i: (i, 0)),
            pl.BlockSpec((F, F), lambda i: (0, 0)),
            pl.BlockSpec((F, F), lambda i: (0, 0)),
        ],
        out_specs=[
            pl.BlockSpec((1000, F), lambda i: (i, 0)),
            pl.BlockSpec((1000, F), lambda i: (i, 0)),
        ],
        out_shape=[
            jax.ShapeDtypeStruct((N_NODES, F), jnp.float32),
            jax.ShapeDtypeStruct((N_NODES, F), jnp.float32),
        ],
    )(x, W, a2)


def _combine_body(aggp_ref, den_ref, agg_ref, dout_ref):
    d = den_ref[...]
    dout_ref[...] = d
    agg_ref[...] = aggp_ref[...] / d


def _combine_stage(aggp, den):
    grid = N_NODES // 1000
    return pl.pallas_call(
        _combine_body,
        grid=(grid,),
        in_specs=[
            pl.BlockSpec((1000, F), lambda i: (i, 0)),
            pl.BlockSpec((1000, 1), lambda i: (i, 0)),
        ],
        out_specs=[
            pl.BlockSpec((1000, F), lambda i: (i, 0)),
            pl.BlockSpec((1000, 1), lambda i: (i, 0)),
        ],
        out_shape=[
            jax.ShapeDtypeStruct((N_NODES, F), jnp.float32),
            jax.ShapeDtypeStruct((N_NODES, 1), jnp.float32),
        ],
    )(aggp, den)


def _sc_body(act_hbm, sd_hbm, ss_hbm, src_hbm, dst_hbm,
             msg_hbm, w_hbm, aggp_hbm, denp_hbm,
             sdst_t, ssrc_t, src_a, src_b, dst_a, dst_b,
             dstl_a, dstl_b, rows, wrows, wbuf, idx_z, agg_s, den_s, sem):
    sid = lax.axis_index("s")

    def _set_idx_z(j, stride):
        for g in range(4):
            idx_z[pl.ds(g * 16, 16)] = (
                jnp.full((16,), sid * stride + j * 64 + g * 16, jnp.int32)
                + lax.iota(jnp.int32, 16))

    pltpu.sync_copy(sd_hbm, sdst_t)
    pltpu.sync_copy(ss_hbm, ssrc_t)

    def _zw(g, carry):
        wbuf[pl.ds(g * 16, 16)] = jnp.zeros((16,), jnp.float32)
        return carry

    lax.fori_loop(0, STRIDE // 16, _zw, None)

    def _zrow(r, carry):
        for f in range(F // 16):
            wrows[r, pl.ds(f * 16, 16)] = jnp.zeros((16,), jnp.float32)
        return carry

    for ph in range(2):
        nbase = ph * NH
        lax.fori_loop(0, 64, _zrow, None)
        for j in range(5):
            _set_idx_z(j, ROWS_T)
            pltpu.sync_copy(wrows.at[pl.ds(0, 64)], agg_s.at[idx_z])
        if ph == 0:
            for j in range(10):
                _set_idx_z(j, STRIDE)
                pltpu.sync_copy(wbuf.at[pl.ds(0, 64)], den_s.at[idx_z])
        plsc.subcore_barrier()

        def _chunk(k, carry):
            c = sid + k * 16

            @pl.when(c < CHUNKS)
            def _():
                base = c * CHUNK
                pltpu.sync_copy(src_hbm.at[pl.ds(base, 128)], src_a)
                pltpu.sync_copy(src_hbm.at[pl.ds(base + 128, 112)], src_b)
                pltpu.sync_copy(dst_hbm.at[pl.ds(base, 128)], dst_a)
                pltpu.sync_copy(dst_hbm.at[pl.ds(base + 128, 112)], dst_b)
                if ph == 0:
                    cp_a = pltpu.async_copy(act_hbm.at[src_a],
                                            rows.at[pl.ds(0, 128)], sem)
                    cp_b = pltpu.async_copy(act_hbm.at[src_b],
                                            rows.at[pl.ds(128, 112)], sem)
                    cp_a.wait()
                    cp_b.wait()
                    pltpu.sync_copy(rows, msg_hbm.at[pl.ds(base, CHUNK)])

                    for g in range(CHUNK // 16):
                        if g < 8:
                            dv = dst_a[pl.ds(g * 16, 16)]
                            sv = src_a[pl.ds(g * 16, 16)]
                        else:
                            dv = dst_b[pl.ds((g - 8) * 16, 16)]
                            sv = src_b[pl.ds((g - 8) * 16, 16)]
                        e = (plsc.load_gather(sdst_t, [dv])
                             + plsc.load_gather(ssrc_t, [sv]))
                        e = jnp.where(e > 0.0, e, e * 0.2)
                        wbuf[pl.ds(g * 16, 16)] = jnp.exp(e)

                    pltpu.sync_copy(wbuf.at[pl.ds(0, CHUNK)],
                                    w_hbm.at[pl.ds(base, CHUNK)])
                else:
                    pltpu.sync_copy(msg_hbm.at[pl.ds(base, CHUNK)], rows)
                    pltpu.sync_copy(w_hbm.at[pl.ds(base, CHUNK)],
                                    wbuf.at[pl.ds(0, CHUNK)])

                for g in range(CHUNK // 16):
                    if g < 8:
                        dv = dst_a[pl.ds(g * 16, 16)]
                    else:
                        dv = dst_b[pl.ds((g - 8) * 16, 16)]
                    ld = dv - nbase
                    ok = (ld >= 0) & (ld < NH)
                    ld = jnp.where(ok, ld, NH)
                    if g < 8:
                        dstl_a[pl.ds(g * 16, 16)] = ld
                    else:
                        dstl_b[pl.ds((g - 8) * 16, 16)] = ld

                def _mul_row(r, carry2):
                    wsp = plsc.load_gather(wbuf, [jnp.full((16,), r, jnp.int32)])
                    for f in range(F // 16):
                        sl = pl.ds(f * 16, 16)
                        wrows[r, sl] = rows[r, sl] * wsp
                    return carry2

                lax.fori_loop(0, CHUNK, _mul_row, None)
                pltpu.sync_copy(wrows.at[pl.ds(0, 128)], agg_s.at[dstl_a],
                                add=True)
                pltpu.sync_copy(wrows.at[pl.ds(128, 112)], agg_s.at[dstl_b],
                                add=True)
                if ph == 0:
                    pltpu.sync_copy(wbuf.at[pl.ds(0, 128)], den_s.at[dst_a],
                                    add=True)
                    pltpu.sync_copy(wbuf.at[pl.ds(128, 112)], den_s.at[dst_b],
                                    add=True)

            return carry

        lax.fori_loop(0, K_ITERS, _chunk, None)
        plsc.subcore_barrier()

        for j in range(5):
            _set_idx_z(j, ROWS_T)
            pltpu.sync_copy(agg_s.at[idx_z], wrows.at[pl.ds(0, 64)])
            pltpu.sync_copy(
                wrows.at[pl.ds(0, 64)],
                aggp_hbm.at[ph, pl.ds(sid * ROWS_T + j * 64, 64)])
        if ph == 0:
            for j in range(10):
                _set_idx_z(j, STRIDE)
                pltpu.sync_copy(den_s.at[idx_z], wbuf.at[pl.ds(0, 64)])
                pltpu.sync_copy(
                    wbuf.at[pl.ds(0, 64)],
                    denp_hbm.at[pl.ds(sid * STRIDE + j * 64, 64)])


def _edge_stage(act, sd, ss, src, dst):
    mesh = plsc.VectorSubcoreMesh(core_axis_name="c", subcore_axis_name="s",
                                  num_cores=1)
    fn = pl.kernel(
        _sc_body,
        mesh=mesh,
        compiler_params=pltpu.CompilerParams(needs_layout_passes=False),
        out_type=[
            jax.ShapeDtypeStruct((E_TOT, F), jnp.float32),
            jax.ShapeDtypeStruct((E_TOT,), jnp.float32),
            jax.ShapeDtypeStruct((2, NH, F), jnp.float32),
            jax.ShapeDtypeStruct((N_PAD,), jnp.float32),
        ],
        scratch_types=[
            pltpu.VMEM((N_NODES,), jnp.float32),
            pltpu.VMEM((N_NODES,), jnp.float32),
            pltpu.VMEM((128,), jnp.int32),
            pltpu.VMEM((112,), jnp.int32),
            pltpu.VMEM((128,), jnp.int32),
            pltpu.VMEM((112,), jnp.int32),
            pltpu.VMEM((128,), jnp.int32),
            pltpu.VMEM((112,), jnp.int32),
            pltpu.VMEM((CHUNK, F), jnp.float32),
            pltpu.VMEM((CHUNK, F), jnp.float32),
            pltpu.VMEM((STRIDE,), jnp.float32),
            pltpu.VMEM((64,), jnp.int32),
            pltpu.VMEM_SHARED((AGG_ROWS, F), jnp.float32),
            pltpu.VMEM_SHARED((N_PAD,), jnp.float32),
            pltpu.SemaphoreType.DMA,
        ],
    )
    return fn(act, sd, ss, src, dst)


def kernel(x, edge_index, W, a):
    ei = edge_index.astype(jnp.int32)
    loops = jnp.arange(N_NODES, dtype=jnp.int32)
    src = jnp.concatenate([ei[0], loops])
    dst = jnp.concatenate([ei[1], loops])

    a2 = jnp.zeros((F, F), jnp.float32)
    a2 = a2.at[:, 0].set(a[:F]).at[:, 1].set(a[F:])
    act, s_pair = _node_stage(x, W, a2)
    sd = s_pair[:, 0]
    ss = s_pair[:, 1]

    msg, w, aggp, denp = _edge_stage(act, sd, ss, src, dst)

    den = denp[:N_NODES].reshape(N_NODES, 1)
    agg, dout = _combine_stage(aggp.reshape(2 * NH, F), den)
    return (agg, w, dout.reshape(N_NODES), msg)


# speedup vs baseline: 2.7433x; 1.9280x over previous
import functools

import jax
import jax.numpy as jnp
from jax import lax
from jax.experimental import pallas as pl
from jax.experimental.pallas import tpu as pltpu
from jax.experimental.pallas import tpu_sc as plsc

N_NODES = 10000
F = 128
FH = 64
N_EDGES = 320000
E_TOT = N_EDGES + N_NODES
CHUNK = 240
CHUNKS = E_TOT // CHUNK
K_ITERS = -(-CHUNKS // 16)
N_PAD = 10240
STRIDE = 640
NH = 5120
W_SPLIT = -(-CHUNKS // 2)
ROWS_T = NH // 16
AGG_ROWS = NH + 8


def _mm_body(x_ref, w_ref, a2_ref, act_ref, sp_ref):
    act = lax.dot_general(x_ref[...], w_ref[...], (((1,), (1,)), ((), ())),
                          preferred_element_type=jnp.float32)
    act_ref[...] = act
    sp_ref[...] = lax.dot_general(act, a2_ref[...], (((1,), (0,)), ((), ())),
                                  preferred_element_type=jnp.float32)


def _node_stage(x, W, a2):
    grid = N_NODES // 1000
    return pl.pallas_call(
        _mm_body,
        grid=(grid,),
        in_specs=[
            pl.BlockSpec((1000, F), lambda i: (i, 0)),
            pl.BlockSpec((F, F), lambda i: (0, 0)),
            pl.BlockSpec((F, F), lambda i: (0, 0)),
        ],
        out_specs=[
            pl.BlockSpec((1000, F), lambda i: (i, 0)),
            pl.BlockSpec((1000, F), lambda i: (i, 0)),
        ],
        out_shape=[
            jax.ShapeDtypeStruct((N_NODES, F), jnp.float32),
            jax.ShapeDtypeStruct((N_NODES, F), jnp.float32),
        ],
    )(x, W, a2)


def _combine_body(aggp_ref, den_ref, agg_ref, dout_ref):
    d = den_ref[...]
    dout_ref[...] = d
    agg_ref[...] = aggp_ref[...] / d


def _combine_stage(aggp, den):
    grid = N_NODES // 1000
    return pl.pallas_call(
        _combine_body,
        grid=(grid,),
        in_specs=[
            pl.BlockSpec((1000, F), lambda i: (i, 0)),
            pl.BlockSpec((1000, 1), lambda i: (i, 0)),
        ],
        out_specs=[
            pl.BlockSpec((1000, F), lambda i: (i, 0)),
            pl.BlockSpec((1000, 1), lambda i: (i, 0)),
        ],
        out_shape=[
            jax.ShapeDtypeStruct((N_NODES, F), jnp.float32),
            jax.ShapeDtypeStruct((N_NODES, 1), jnp.float32),
        ],
    )(aggp, den)


def _sc_body(act_hbm, sd_hbm, ss_hbm, src_hbm, dst_hbm,
             msg_hbm, w_hbm, aggp_hbm, denp_hbm,
             sdst_t, ssrc_t, src_a, src_b, dst_a, dst_b,
             dstl_a, dstl_b, rows, wrows, wbuf, idx_z, agg_s, den_s, sem):
    sid = lax.axis_index("s")
    cid = lax.axis_index("c")
    nbase = cid * NH

    def _set_idx_z(j, stride):
        for g in range(4):
            idx_z[pl.ds(g * 16, 16)] = (
                jnp.full((16,), sid * stride + j * 64 + g * 16, jnp.int32)
                + lax.iota(jnp.int32, 16))

    pltpu.sync_copy(sd_hbm, sdst_t)
    pltpu.sync_copy(ss_hbm, ssrc_t)

    def _zw(g, carry):
        wbuf[pl.ds(g * 16, 16)] = jnp.zeros((16,), jnp.float32)
        return carry

    lax.fori_loop(0, STRIDE // 16, _zw, None)

    def _zrow(r, carry):
        for f in range(F // 16):
            wrows[r, pl.ds(f * 16, 16)] = jnp.zeros((16,), jnp.float32)
        return carry

    lax.fori_loop(0, 64, _zrow, None)
    for j in range(5):
        _set_idx_z(j, ROWS_T)
        pltpu.sync_copy(wrows.at[pl.ds(0, 64)], agg_s.at[idx_z])

    @pl.when(cid == 0)
    def _den_zero():
        for j in range(10):
            _set_idx_z(j, STRIDE)
            pltpu.sync_copy(wbuf.at[pl.ds(0, 64)], den_s.at[idx_z])

    plsc.subcore_barrier()

    def _chunk(k, carry):
        c = sid + k * 16

        @pl.when(c < CHUNKS)
        def _():
            base = c * CHUNK
            pltpu.sync_copy(src_hbm.at[pl.ds(base, 128)], src_a)
            pltpu.sync_copy(src_hbm.at[pl.ds(base + 128, 112)], src_b)
            pltpu.sync_copy(dst_hbm.at[pl.ds(base, 128)], dst_a)
            pltpu.sync_copy(dst_hbm.at[pl.ds(base + 128, 112)], dst_b)
            cp_a = pltpu.async_copy(act_hbm.at[src_a],
                                    rows.at[pl.ds(0, 128)], sem)
            cp_b = pltpu.async_copy(act_hbm.at[src_b],
                                    rows.at[pl.ds(128, 112)], sem)
            cp_a.wait()
            cp_b.wait()

            for g in range(CHUNK // 16):
                if g < 8:
                    dv = dst_a[pl.ds(g * 16, 16)]
                    sv = src_a[pl.ds(g * 16, 16)]
                else:
                    dv = dst_b[pl.ds((g - 8) * 16, 16)]
                    sv = src_b[pl.ds((g - 8) * 16, 16)]
                e = (plsc.load_gather(sdst_t, [dv])
                     + plsc.load_gather(ssrc_t, [sv]))
                e = jnp.where(e > 0.0, e, e * 0.2)
                wbuf[pl.ds(g * 16, 16)] = jnp.exp(e)

            @pl.when((c < W_SPLIT) == (cid == 0))
            def _own_out():
                pltpu.sync_copy(rows, msg_hbm.at[pl.ds(base, CHUNK)])
                pltpu.sync_copy(wbuf.at[pl.ds(0, CHUNK)],
                                w_hbm.at[pl.ds(base, CHUNK)])

            for g in range(CHUNK // 16):
                if g < 8:
                    dv = dst_a[pl.ds(g * 16, 16)]
                else:
                    dv = dst_b[pl.ds((g - 8) * 16, 16)]
                ld = dv - nbase
                ok = (ld >= 0) & (ld < NH)
                ld = jnp.where(ok, ld, NH)
                if g < 8:
                    dstl_a[pl.ds(g * 16, 16)] = ld
                else:
                    dstl_b[pl.ds((g - 8) * 16, 16)] = ld

            def _mul_row(r, carry2):
                wsp = plsc.load_gather(wbuf, [jnp.full((16,), r, jnp.int32)])
                for f in range(F // 16):
                    sl = pl.ds(f * 16, 16)
                    wrows[r, sl] = rows[r, sl] * wsp
                return carry2

            lax.fori_loop(0, CHUNK, _mul_row, None)
            pltpu.sync_copy(wrows.at[pl.ds(0, 128)], agg_s.at[dstl_a],
                            add=True)
            pltpu.sync_copy(wrows.at[pl.ds(128, 112)], agg_s.at[dstl_b],
                            add=True)

            @pl.when(cid == 0)
            def _den_add():
                pltpu.sync_copy(wbuf.at[pl.ds(0, 128)], den_s.at[dst_a],
                                add=True)
                pltpu.sync_copy(wbuf.at[pl.ds(128, 112)], den_s.at[dst_b],
                                add=True)

        return carry

    lax.fori_loop(0, K_ITERS, _chunk, None)
    plsc.subcore_barrier()

    for j in range(5):
        _set_idx_z(j, ROWS_T)
        pltpu.sync_copy(agg_s.at[idx_z], wrows.at[pl.ds(0, 64)])
        pltpu.sync_copy(
            wrows.at[pl.ds(0, 64)],
            aggp_hbm.at[cid, pl.ds(sid * ROWS_T + j * 64, 64)])

    @pl.when(cid == 0)
    def _den_out():
        for j in range(10):
            _set_idx_z(j, STRIDE)
            pltpu.sync_copy(den_s.at[idx_z], wbuf.at[pl.ds(0, 64)])
            pltpu.sync_copy(
                wbuf.at[pl.ds(0, 64)],
                denp_hbm.at[pl.ds(sid * STRIDE + j * 64, 64)])


def _edge_stage(act, sd, ss, src, dst):
    mesh = plsc.VectorSubcoreMesh(core_axis_name="c", subcore_axis_name="s",
                                  num_cores=2)
    fn = pl.kernel(
        _sc_body,
        mesh=mesh,
        compiler_params=pltpu.CompilerParams(needs_layout_passes=False),
        out_type=[
            jax.ShapeDtypeStruct((E_TOT, F), jnp.float32),
            jax.ShapeDtypeStruct((E_TOT,), jnp.float32),
            jax.ShapeDtypeStruct((2, NH, F), jnp.float32),
            jax.ShapeDtypeStruct((N_PAD,), jnp.float32),
        ],
        scratch_types=[
            pltpu.VMEM((N_NODES,), jnp.float32),
            pltpu.VMEM((N_NODES,), jnp.float32),
            pltpu.VMEM((128,), jnp.int32),
            pltpu.VMEM((112,), jnp.int32),
            pltpu.VMEM((128,), jnp.int32),
            pltpu.VMEM((112,), jnp.int32),
            pltpu.VMEM((128,), jnp.int32),
            pltpu.VMEM((112,), jnp.int32),
            pltpu.VMEM((CHUNK, F), jnp.float32),
            pltpu.VMEM((CHUNK, F), jnp.float32),
            pltpu.VMEM((STRIDE,), jnp.float32),
            pltpu.VMEM((64,), jnp.int32),
            pltpu.VMEM_SHARED((AGG_ROWS, F), jnp.float32),
            pltpu.VMEM_SHARED((N_PAD,), jnp.float32),
            pltpu.SemaphoreType.DMA,
        ],
    )
    return fn(act, sd, ss, src, dst)


def kernel(x, edge_index, W, a):
    ei = edge_index.astype(jnp.int32)
    loops = jnp.arange(N_NODES, dtype=jnp.int32)
    src = jnp.concatenate([ei[0], loops])
    dst = jnp.concatenate([ei[1], loops])

    a2 = jnp.zeros((F, F), jnp.float32)
    a2 = a2.at[:, 0].set(a[:F]).at[:, 1].set(a[F:])
    act, s_pair = _node_stage(x, W, a2)
    sd = s_pair[:, 0]
    ss = s_pair[:, 1]

    msg, w, aggp, denp = _edge_stage(act, sd, ss, src, dst)

    den = denp[:N_NODES].reshape(N_NODES, 1)
    agg, dout = _combine_stage(aggp.reshape(2 * NH, F), den)
    return (agg, w, dout.reshape(N_NODES), msg)


# speedup vs baseline: 6.9864x; 2.5467x over previous
import functools

import jax
import jax.numpy as jnp
from jax import lax
from jax.experimental import pallas as pl
from jax.experimental.pallas import tpu as pltpu
from jax.experimental.pallas import tpu_sc as plsc

N_NODES = 10000
F = 128
FH = 64
N_EDGES = 320000
E_TOT = N_EDGES + N_NODES
CHUNK = 240
CHUNKS = E_TOT // CHUNK
K_ITERS = -(-CHUNKS // 16)
N_PAD = 10240
STRIDE = 640
NH = 5120
W_SPLIT = -(-CHUNKS // 2)
ROWS_T = NH // 16
AGG_ROWS = NH + 8


def _mm_body(x_ref, w_ref, a2_ref, act_ref, sp_ref):
    act = lax.dot_general(x_ref[...], w_ref[...], (((1,), (1,)), ((), ())),
                          preferred_element_type=jnp.float32)
    act_ref[...] = act
    sp_ref[...] = lax.dot_general(act, a2_ref[...], (((1,), (0,)), ((), ())),
                                  preferred_element_type=jnp.float32)


def _node_stage(x, W, a2):
    grid = N_NODES // 1000
    return pl.pallas_call(
        _mm_body,
        grid=(grid,),
        in_specs=[
            pl.BlockSpec((1000, F), lambda i: (i, 0)),
            pl.BlockSpec((F, F), lambda i: (0, 0)),
            pl.BlockSpec((F, F), lambda i: (0, 0)),
        ],
        out_specs=[
            pl.BlockSpec((1000, F), lambda i: (i, 0)),
            pl.BlockSpec((1000, F), lambda i: (i, 0)),
        ],
        out_shape=[
            jax.ShapeDtypeStruct((N_NODES, F), jnp.float32),
            jax.ShapeDtypeStruct((N_NODES, F), jnp.float32),
        ],
    )(x, W, a2)


def _combine_body(aggp_ref, den_ref, agg_ref, dout_ref):
    d = den_ref[...]
    dout_ref[...] = d
    agg_ref[...] = aggp_ref[...] / d


def _combine_stage(aggp, den):
    grid = N_NODES // 1000
    return pl.pallas_call(
        _combine_body,
        grid=(grid,),
        in_specs=[
            pl.BlockSpec((1000, F), lambda i: (i, 0)),
            pl.BlockSpec((1000, 1), lambda i: (i, 0)),
        ],
        out_specs=[
            pl.BlockSpec((1000, F), lambda i: (i, 0)),
            pl.BlockSpec((1000, 1), lambda i: (i, 0)),
        ],
        out_shape=[
            jax.ShapeDtypeStruct((N_NODES, F), jnp.float32),
            jax.ShapeDtypeStruct((N_NODES, 1), jnp.float32),
        ],
    )(aggp, den)


def _sc_body(act_hbm, sd_hbm, ss_hbm, src_hbm, dst_hbm,
             msg_hbm, w_hbm, aggp_hbm, denp_hbm,
             sdst_t, ssrc_t, src_a, src_b, dst_a, dst_b,
             dstl_a, dstl_b, rows, wrows, wbuf, idx_z, agg_s, den_s, sem,
             sem2):
    sid = lax.axis_index("s")
    cid = lax.axis_index("c")
    nbase = cid * NH

    def _set_idx_z(j, stride):
        for g in range(4):
            idx_z[pl.ds(g * 16, 16)] = (
                jnp.full((16,), sid * stride + j * 64 + g * 16, jnp.int32)
                + lax.iota(jnp.int32, 16))

    pltpu.sync_copy(sd_hbm, sdst_t)
    pltpu.sync_copy(ss_hbm, ssrc_t)

    def _zw(g, carry):
        wbuf[pl.ds(g * 16, 16)] = jnp.zeros((16,), jnp.float32)
        return carry

    lax.fori_loop(0, STRIDE // 16, _zw, None)

    def _zrow(r, carry):
        for f in range(F // 16):
            wrows[r, pl.ds(f * 16, 16)] = jnp.zeros((16,), jnp.float32)
        return carry

    lax.fori_loop(0, 64, _zrow, None)
    for j in range(5):
        _set_idx_z(j, ROWS_T)
        pltpu.sync_copy(wrows.at[pl.ds(0, 64)], agg_s.at[idx_z])

    @pl.when(cid == 0)
    def _den_zero():
        for j in range(10):
            _set_idx_z(j, STRIDE)
            pltpu.sync_copy(wbuf.at[pl.ds(0, 64)], den_s.at[idx_z])

    plsc.subcore_barrier()

    def _chunk(k, carry):
        c = sid + k * 16

        @pl.when(c < CHUNKS)
        def _():
            base = c * CHUNK
            owner = (c < W_SPLIT) == (cid == 0)
            i1 = pltpu.async_copy(src_hbm.at[pl.ds(base, 128)], src_a, sem)
            i2 = pltpu.async_copy(src_hbm.at[pl.ds(base + 128, 112)], src_b,
                                  sem)
            i3 = pltpu.async_copy(dst_hbm.at[pl.ds(base, 128)], dst_a, sem)
            i4 = pltpu.async_copy(dst_hbm.at[pl.ds(base + 128, 112)], dst_b,
                                  sem)
            i1.wait()
            i2.wait()
            i3.wait()
            i4.wait()
            cp_a = pltpu.async_copy(act_hbm.at[src_a],
                                    rows.at[pl.ds(0, 128)], sem)
            cp_b = pltpu.async_copy(act_hbm.at[src_b],
                                    rows.at[pl.ds(128, 112)], sem)

            for g in range(CHUNK // 16):
                if g < 8:
                    dv = dst_a[pl.ds(g * 16, 16)]
                    sv = src_a[pl.ds(g * 16, 16)]
                else:
                    dv = dst_b[pl.ds((g - 8) * 16, 16)]
                    sv = src_b[pl.ds((g - 8) * 16, 16)]
                e = (plsc.load_gather(sdst_t, [dv])
                     + plsc.load_gather(ssrc_t, [sv]))
                e = jnp.where(e > 0.0, e, e * 0.2)
                wbuf[pl.ds(g * 16, 16)] = jnp.exp(e)
                ld = dv - nbase
                ok = (ld >= 0) & (ld < NH)
                ld = jnp.where(ok, ld, NH)
                if g < 8:
                    dstl_a[pl.ds(g * 16, 16)] = ld
                else:
                    dstl_b[pl.ds((g - 8) * 16, 16)] = ld

            cp_a.wait()
            cp_b.wait()

            @pl.when(owner)
            def _own_out():
                pltpu.async_copy(rows, msg_hbm.at[pl.ds(base, CHUNK)], sem2)
                pltpu.async_copy(wbuf.at[pl.ds(0, CHUNK)],
                                 w_hbm.at[pl.ds(base, CHUNK)], sem2)

            @plsc.parallel_loop(0, CHUNK, 1, unroll=4)
            def _mul_row(r):
                wsp = plsc.load_gather(wbuf, [jnp.full((16,), r, jnp.int32)])
                for f in range(F // 16):
                    sl = pl.ds(f * 16, 16)
                    wrows[r, sl] = rows[r, sl] * wsp

            @pl.when(owner)
            def _own_drain():
                pltpu.make_async_copy(
                    rows, msg_hbm.at[pl.ds(base, CHUNK)], sem2).wait()
                pltpu.make_async_copy(
                    wbuf.at[pl.ds(0, CHUNK)],
                    w_hbm.at[pl.ds(base, CHUNK)], sem2).wait()

            s1 = pltpu.async_copy(wrows.at[pl.ds(0, 128)], agg_s.at[dstl_a],
                                  sem, add=True)
            s2 = pltpu.async_copy(wrows.at[pl.ds(128, 112)], agg_s.at[dstl_b],
                                  sem, add=True)

            @pl.when(cid == 0)
            def _den_add():
                pltpu.async_copy(wbuf.at[pl.ds(0, 128)], den_s.at[dst_a],
                                 sem2, add=True)
                pltpu.async_copy(wbuf.at[pl.ds(128, 112)], den_s.at[dst_b],
                                 sem2, add=True)

            s1.wait()
            s2.wait()

            @pl.when(cid == 0)
            def _den_drain():
                pltpu.make_async_copy(
                    wbuf.at[pl.ds(0, 128)], den_s.at[dst_a], sem2).wait()
                pltpu.make_async_copy(
                    wbuf.at[pl.ds(128, 112)], den_s.at[dst_b], sem2).wait()

        return carry

    lax.fori_loop(0, K_ITERS, _chunk, None)
    plsc.subcore_barrier()

    for j in range(5):
        _set_idx_z(j, ROWS_T)
        pltpu.sync_copy(agg_s.at[idx_z], wrows.at[pl.ds(0, 64)])
        pltpu.sync_copy(
            wrows.at[pl.ds(0, 64)],
            aggp_hbm.at[cid, pl.ds(sid * ROWS_T + j * 64, 64)])

    @pl.when(cid == 0)
    def _den_out():
        for j in range(10):
            _set_idx_z(j, STRIDE)
            pltpu.sync_copy(den_s.at[idx_z], wbuf.at[pl.ds(0, 64)])
            pltpu.sync_copy(
                wbuf.at[pl.ds(0, 64)],
                denp_hbm.at[pl.ds(sid * STRIDE + j * 64, 64)])


def _edge_stage(act, sd, ss, src, dst):
    mesh = plsc.VectorSubcoreMesh(core_axis_name="c", subcore_axis_name="s",
                                  num_cores=2)
    fn = pl.kernel(
        _sc_body,
        mesh=mesh,
        compiler_params=pltpu.CompilerParams(needs_layout_passes=False),
        out_type=[
            jax.ShapeDtypeStruct((E_TOT, F), jnp.float32),
            jax.ShapeDtypeStruct((E_TOT,), jnp.float32),
            jax.ShapeDtypeStruct((2, NH, F), jnp.float32),
            jax.ShapeDtypeStruct((N_PAD,), jnp.float32),
        ],
        scratch_types=[
            pltpu.VMEM((N_NODES,), jnp.float32),
            pltpu.VMEM((N_NODES,), jnp.float32),
            pltpu.VMEM((128,), jnp.int32),
            pltpu.VMEM((112,), jnp.int32),
            pltpu.VMEM((128,), jnp.int32),
            pltpu.VMEM((112,), jnp.int32),
            pltpu.VMEM((128,), jnp.int32),
            pltpu.VMEM((112,), jnp.int32),
            pltpu.VMEM((CHUNK, F), jnp.float32),
            pltpu.VMEM((CHUNK, F), jnp.float32),
            pltpu.VMEM((STRIDE,), jnp.float32),
            pltpu.VMEM((64,), jnp.int32),
            pltpu.VMEM_SHARED((AGG_ROWS, F), jnp.float32),
            pltpu.VMEM_SHARED((N_PAD,), jnp.float32),
            pltpu.SemaphoreType.DMA,
            pltpu.SemaphoreType.DMA,
        ],
    )
    return fn(act, sd, ss, src, dst)


def kernel(x, edge_index, W, a):
    ei = edge_index.astype(jnp.int32)
    loops = jnp.arange(N_NODES, dtype=jnp.int32)
    src = jnp.concatenate([ei[0], loops])
    dst = jnp.concatenate([ei[1], loops])

    a2 = jnp.zeros((F, F), jnp.float32)
    a2 = a2.at[:, 0].set(a[:F]).at[:, 1].set(a[F:])
    act, s_pair = _node_stage(x, W, a2)
    sd = s_pair[:, 0]
    ss = s_pair[:, 1]

    msg, w, aggp, denp = _edge_stage(act, sd, ss, src, dst)

    den = denp[:N_NODES].reshape(N_NODES, 1)
    agg, dout = _combine_stage(aggp.reshape(2 * NH, F), den)
    return (agg, w, dout.reshape(N_NODES), msg)


# speedup vs baseline: 7.4770x; 1.0702x over previous
import functools

import jax
import jax.numpy as jnp
from jax import lax
from jax.experimental import pallas as pl
from jax.experimental.pallas import tpu as pltpu
from jax.experimental.pallas import tpu_sc as plsc

N_NODES = 10000
F = 128
FH = 64
N_EDGES = 320000
E_TOT = N_EDGES + N_NODES
CHUNK = 240
CHUNKS = E_TOT // CHUNK
K_ITERS = -(-CHUNKS // 16)
N_PAD = 10240
STRIDE = 640
NH = 5120
W_SPLIT = -(-CHUNKS // 2)
ROWS_T = NH // 16
AGG_ROWS = NH + 8


def _mm_body(x_ref, w_ref, a2_ref, act_ref, sp_ref):
    act = lax.dot_general(x_ref[...], w_ref[...], (((1,), (1,)), ((), ())),
                          preferred_element_type=jnp.float32)
    act_ref[...] = act
    sp_ref[...] = lax.dot_general(act, a2_ref[...], (((1,), (0,)), ((), ())),
                                  preferred_element_type=jnp.float32)


def _node_stage(x, W, a2):
    grid = N_NODES // 1000
    return pl.pallas_call(
        _mm_body,
        grid=(grid,),
        in_specs=[
            pl.BlockSpec((1000, F), lambda i: (i, 0)),
            pl.BlockSpec((F, F), lambda i: (0, 0)),
            pl.BlockSpec((F, F), lambda i: (0, 0)),
        ],
        out_specs=[
            pl.BlockSpec((1000, F), lambda i: (i, 0)),
            pl.BlockSpec((1000, F), lambda i: (i, 0)),
        ],
        out_shape=[
            jax.ShapeDtypeStruct((N_NODES, F), jnp.float32),
            jax.ShapeDtypeStruct((N_NODES, F), jnp.float32),
        ],
    )(x, W, a2)


def _combine_body(aggp_ref, den_ref, agg_ref, dout_ref):
    d = den_ref[...]
    dout_ref[...] = d
    agg_ref[...] = aggp_ref[...] / d


def _combine_stage(aggp, den):
    grid = N_NODES // 1000
    return pl.pallas_call(
        _combine_body,
        grid=(grid,),
        in_specs=[
            pl.BlockSpec((1000, F), lambda i: (i, 0)),
            pl.BlockSpec((1000, 1), lambda i: (i, 0)),
        ],
        out_specs=[
            pl.BlockSpec((1000, F), lambda i: (i, 0)),
            pl.BlockSpec((1000, 1), lambda i: (i, 0)),
        ],
        out_shape=[
            jax.ShapeDtypeStruct((N_NODES, F), jnp.float32),
            jax.ShapeDtypeStruct((N_NODES, 1), jnp.float32),
        ],
    )(aggp, den)


def _sc_body(act_hbm, sd_hbm, ss_hbm, src_hbm, dst_hbm,
             msg_hbm, w_hbm, aggp_hbm, denp_hbm,
             sdst_t, ssrc_t, src_a, src_b, dst_a, dst_b,
             dstl_a, dstl_b, rows, wrows, wbuf, idx_z,
             src_a2, src_b2, dst_a2, dst_b2,
             agg_s, den_s, sem, sem2, sem_i0, sem_i1):
    sid = lax.axis_index("s")
    cid = lax.axis_index("c")
    nbase = cid * NH

    def _set_idx_z(j, stride):
        for g in range(4):
            idx_z[pl.ds(g * 16, 16)] = (
                jnp.full((16,), sid * stride + j * 64 + g * 16, jnp.int32)
                + lax.iota(jnp.int32, 16))

    pltpu.sync_copy(sd_hbm, sdst_t)
    pltpu.sync_copy(ss_hbm, ssrc_t)

    def _zw(g, carry):
        wbuf[pl.ds(g * 16, 16)] = jnp.zeros((16,), jnp.float32)
        return carry

    lax.fori_loop(0, STRIDE // 16, _zw, None)

    def _zrow(r, carry):
        for f in range(F // 16):
            wrows[r, pl.ds(f * 16, 16)] = jnp.zeros((16,), jnp.float32)
        return carry

    lax.fori_loop(0, 64, _zrow, None)
    for j in range(5):
        _set_idx_z(j, ROWS_T)
        pltpu.sync_copy(wrows.at[pl.ds(0, 64)], agg_s.at[idx_z])

    @pl.when(cid == 0)
    def _den_zero():
        for j in range(10):
            _set_idx_z(j, STRIDE)
            pltpu.sync_copy(wbuf.at[pl.ds(0, 64)], den_s.at[idx_z])

    plsc.subcore_barrier()

    iset0 = (src_a, src_b, dst_a, dst_b, sem_i0)
    iset1 = (src_a2, src_b2, dst_a2, dst_b2, sem_i1)

    def _fire_idx(c, st):
        sa, sb, da, db, semi = st
        base = c * CHUNK
        pltpu.async_copy(src_hbm.at[pl.ds(base, 128)], sa, semi)
        pltpu.async_copy(src_hbm.at[pl.ds(base + 128, 112)], sb, semi)
        pltpu.async_copy(dst_hbm.at[pl.ds(base, 128)], da, semi)
        pltpu.async_copy(dst_hbm.at[pl.ds(base + 128, 112)], db, semi)

    def _drain_idx(c, st):
        sa, sb, da, db, semi = st
        base = c * CHUNK
        pltpu.make_async_copy(src_hbm.at[pl.ds(base, 128)], sa, semi).wait()
        pltpu.make_async_copy(src_hbm.at[pl.ds(base + 128, 112)], sb,
                              semi).wait()
        pltpu.make_async_copy(dst_hbm.at[pl.ds(base, 128)], da, semi).wait()
        pltpu.make_async_copy(dst_hbm.at[pl.ds(base + 128, 112)], db,
                              semi).wait()

    def _step(k, cur, nxt):
        c = sid + k * 16

        @pl.when(c < CHUNKS)
        def _():
            sa_c, sb_c, da_c, db_c, _ = cur
            base = c * CHUNK
            owner = (c < W_SPLIT) == (cid == 0)
            _drain_idx(c, cur)
            cp_a = pltpu.async_copy(act_hbm.at[sa_c],
                                    rows.at[pl.ds(0, 128)], sem)
            cp_b = pltpu.async_copy(act_hbm.at[sb_c],
                                    rows.at[pl.ds(128, 112)], sem)

            @pl.when(c + 16 < CHUNKS)
            def _prefetch_idx():
                _fire_idx(c + 16, nxt)

            for g in range(CHUNK // 16):
                if g < 8:
                    dv = da_c[pl.ds(g * 16, 16)]
                    sv = sa_c[pl.ds(g * 16, 16)]
                else:
                    dv = db_c[pl.ds((g - 8) * 16, 16)]
                    sv = sb_c[pl.ds((g - 8) * 16, 16)]
                e = (plsc.load_gather(sdst_t, [dv])
                     + plsc.load_gather(ssrc_t, [sv]))
                e = jnp.where(e > 0.0, e, e * 0.2)
                wbuf[pl.ds(g * 16, 16)] = jnp.exp(e)
                ld = dv - nbase
                ok = (ld >= 0) & (ld < NH)
                ld = jnp.where(ok, ld, NH)
                if g < 8:
                    dstl_a[pl.ds(g * 16, 16)] = ld
                else:
                    dstl_b[pl.ds((g - 8) * 16, 16)] = ld

            cp_a.wait()
            cp_b.wait()

            @pl.when(owner)
            def _own_out():
                pltpu.async_copy(rows, msg_hbm.at[pl.ds(base, CHUNK)], sem2)
                pltpu.async_copy(wbuf.at[pl.ds(0, CHUNK)],
                                 w_hbm.at[pl.ds(base, CHUNK)], sem2)

            @plsc.parallel_loop(0, CHUNK, 1, unroll=4)
            def _mul_row(r):
                wsp = plsc.load_gather(wbuf, [jnp.full((16,), r, jnp.int32)])
                for f in range(F // 16):
                    sl = pl.ds(f * 16, 16)
                    wrows[r, sl] = rows[r, sl] * wsp

            @pl.when(owner)
            def _own_drain():
                pltpu.make_async_copy(
                    rows, msg_hbm.at[pl.ds(base, CHUNK)], sem2).wait()
                pltpu.make_async_copy(
                    wbuf.at[pl.ds(0, CHUNK)],
                    w_hbm.at[pl.ds(base, CHUNK)], sem2).wait()

            s1 = pltpu.async_copy(wrows.at[pl.ds(0, 128)], agg_s.at[dstl_a],
                                  sem, add=True)
            s2 = pltpu.async_copy(wrows.at[pl.ds(128, 112)], agg_s.at[dstl_b],
                                  sem, add=True)

            @pl.when(cid == 0)
            def _den_add():
                pltpu.async_copy(wbuf.at[pl.ds(0, 128)], den_s.at[da_c],
                                 sem2, add=True)
                pltpu.async_copy(wbuf.at[pl.ds(128, 112)], den_s.at[db_c],
                                 sem2, add=True)

            s1.wait()
            s2.wait()

            @pl.when(cid == 0)
            def _den_drain():
                pltpu.make_async_copy(
                    wbuf.at[pl.ds(0, 128)], den_s.at[da_c], sem2).wait()
                pltpu.make_async_copy(
                    wbuf.at[pl.ds(128, 112)], den_s.at[db_c], sem2).wait()

    _fire_idx(sid, iset0)

    def _chunk2(j, carry):
        _step(2 * j, iset0, iset1)
        _step(2 * j + 1, iset1, iset0)
        return carry

    lax.fori_loop(0, K_ITERS // 2, _chunk2, None)
    plsc.subcore_barrier()

    for j in range(5):
        _set_idx_z(j, ROWS_T)
        pltpu.sync_copy(agg_s.at[idx_z], wrows.at[pl.ds(0, 64)])
        pltpu.sync_copy(
            wrows.at[pl.ds(0, 64)],
            aggp_hbm.at[cid, pl.ds(sid * ROWS_T + j * 64, 64)])

    @pl.when(cid == 0)
    def _den_out():
        for j in range(10):
            _set_idx_z(j, STRIDE)
            pltpu.sync_copy(den_s.at[idx_z], wbuf.at[pl.ds(0, 64)])
            pltpu.sync_copy(
                wbuf.at[pl.ds(0, 64)],
                denp_hbm.at[pl.ds(sid * STRIDE + j * 64, 64)])


def _edge_stage(act, sd, ss, src, dst):
    mesh = plsc.VectorSubcoreMesh(core_axis_name="c", subcore_axis_name="s",
                                  num_cores=2)
    fn = pl.kernel(
        _sc_body,
        mesh=mesh,
        compiler_params=pltpu.CompilerParams(needs_layout_passes=False),
        out_type=[
            jax.ShapeDtypeStruct((E_TOT, F), jnp.float32),
            jax.ShapeDtypeStruct((E_TOT,), jnp.float32),
            jax.ShapeDtypeStruct((2, NH, F), jnp.float32),
            jax.ShapeDtypeStruct((N_PAD,), jnp.float32),
        ],
        scratch_types=[
            pltpu.VMEM((N_NODES,), jnp.float32),
            pltpu.VMEM((N_NODES,), jnp.float32),
            pltpu.VMEM((128,), jnp.int32),
            pltpu.VMEM((112,), jnp.int32),
            pltpu.VMEM((128,), jnp.int32),
            pltpu.VMEM((112,), jnp.int32),
            pltpu.VMEM((128,), jnp.int32),
            pltpu.VMEM((112,), jnp.int32),
            pltpu.VMEM((CHUNK, F), jnp.float32),
            pltpu.VMEM((CHUNK, F), jnp.float32),
            pltpu.VMEM((STRIDE,), jnp.float32),
            pltpu.VMEM((64,), jnp.int32),
            pltpu.VMEM((128,), jnp.int32),
            pltpu.VMEM((112,), jnp.int32),
            pltpu.VMEM((128,), jnp.int32),
            pltpu.VMEM((112,), jnp.int32),
            pltpu.VMEM_SHARED((AGG_ROWS, F), jnp.float32),
            pltpu.VMEM_SHARED((N_PAD,), jnp.float32),
            pltpu.SemaphoreType.DMA,
            pltpu.SemaphoreType.DMA,
            pltpu.SemaphoreType.DMA,
            pltpu.SemaphoreType.DMA,
        ],
    )
    return fn(act, sd, ss, src, dst)


def kernel(x, edge_index, W, a):
    ei = edge_index.astype(jnp.int32)
    loops = jnp.arange(N_NODES, dtype=jnp.int32)
    src = jnp.concatenate([ei[0], loops])
    dst = jnp.concatenate([ei[1], loops])

    a2 = jnp.zeros((F, F), jnp.float32)
    a2 = a2.at[:, 0].set(a[:F]).at[:, 1].set(a[F:])
    act, s_pair = _node_stage(x, W, a2)
    sd = s_pair[:, 0]
    ss = s_pair[:, 1]

    msg, w, aggp, denp = _edge_stage(act, sd, ss, src, dst)

    den = denp[:N_NODES].reshape(N_NODES, 1)
    agg, dout = _combine_stage(aggp.reshape(2 * NH, F), den)
    return (agg, w, dout.reshape(N_NODES), msg)
